# Initial kernel scaffold; baseline (speedup 1.0000x reference)
import jax
import jax.numpy as jnp
from jax.experimental import pallas as pl


def kernel(shuf, origin, i, sparse, edge_index, W, b):
    raise NotImplementedError("write your pallas kernel here")


# speedup vs baseline: 3.5734x; 3.5734x over previous
import functools

import jax
import jax.numpy as jnp
from jax.experimental import pallas as pl
from jax.experimental.pallas import tpu as pltpu

_N = 10000
_D = 128
_BM = 1000
_BK = 1024
_KP = 10240
_NKB = _KP // _BK


def _mm_body(origin_ref, oc_ref, out_ref, acc_ref):
    k = pl.program_id(1)
    a = origin_ref[...]
    limit = _N - k * _BK
    col = jax.lax.broadcasted_iota(jnp.int32, a.shape, 1)
    a = jnp.where(col < limit, a, 0.0)
    b = oc_ref[pl.ds(k * _BK, _BK), :]
    part = jnp.dot(a, b, preferred_element_type=jnp.float32)

    @pl.when(k == 0)
    def _():
        acc_ref[...] = jnp.zeros_like(acc_ref)

    acc_ref[...] += part

    @pl.when(k == _NKB - 1)
    def _():
        out_ref[...] = jnp.maximum(acc_ref[...], 0.0)


def _readout_matmul(origin, oc_pad):
    grid = (_N // _BM, _NKB)
    return pl.pallas_call(
        _mm_body,
        grid=grid,
        in_specs=[
            pl.BlockSpec((_BM, _BK), lambda m, k: (m, k)),
            pl.BlockSpec((_KP, _D), lambda m, k: (0, 0)),
        ],
        out_specs=pl.BlockSpec((_BM, _D), lambda m, k: (m, 0)),
        out_shape=jax.ShapeDtypeStruct((_N, _D), jnp.float32),
        scratch_shapes=[pltpu.VMEM((_BM, _D), jnp.float32)],
        compiler_params=pltpu.CompilerParams(
            dimension_semantics=("parallel", "arbitrary"),
        ),
    )(origin, oc_pad)


def kernel(shuf, origin, i, sparse, edge_index, W, b):
    src = edge_index[0]
    dst = edge_index[1]
    n = shuf.shape[0]
    deg = jax.ops.segment_sum(jnp.ones(src.shape[0], jnp.float32), dst,
                              num_segments=n) + 1.0
    dinv = jax.lax.rsqrt(deg)
    g = (shuf @ W) * dinv[:, None]
    acc = jax.ops.segment_sum(g[src], dst, num_segments=n)
    oc = (acc + g) * dinv[:, None] + b[None, :]
    oc_pad = jnp.zeros((_KP, _D), jnp.float32).at[:n].set(oc)
    out = _readout_matmul(origin, oc_pad)
    return out[None]


# speedup vs baseline: 18.6074x; 5.2072x over previous
import functools

import jax
import jax.numpy as jnp
from jax import lax
from jax.experimental import pallas as pl
from jax.experimental.pallas import tpu as pltpu
from jax.experimental.pallas import tpu_sc as plsc

_N = 10000
_D = 128
_H = _D // 2
_E = 320000
_NP = 10240
_NC = 2
_NS = 16
_NW = _NC * _NS
_EW = _E // _NW
_C = 80
_NCH = _EW // _C
_NPAIR = _NCH // 2
_RPT = _NP // _NS
_ZR = 128
_ZB = 64

_BM = 1000
_BK = 1024
_NKB = _NP // _BK
_BN = 1024

_sc_mesh = plsc.VectorSubcoreMesh(core_axis_name="c", subcore_axis_name="s")


def _deg_body(dst_hbm, out_hbm, ia, ib, ones_v, zeros_v, acc_sh, semi):
    c = lax.axis_index("c")
    s = lax.axis_index("s")
    wid = c * _NS + s
    base = wid * _EW

    for j in range(_C // 16):
        ones_v[pl.ds(16 * j, 16)] = jnp.ones((16,), jnp.float32)

    def _zinit(j, carry):
        zeros_v[pl.ds(16 * j, 16)] = jnp.zeros((16,), jnp.float32)
        return carry

    lax.fori_loop(0, _RPT // 16, _zinit, 0)
    pltpu.sync_copy(zeros_v, acc_sh.at[pl.ds(s * _RPT, _RPT)])
    plsc.subcore_barrier()

    pltpu.sync_copy(dst_hbm.at[pl.ds(base, _C)], ia)

    def _pair(p, carry):
        j1 = 2 * p + 1
        pltpu.async_copy(dst_hbm.at[pl.ds(base + j1 * _C, _C)], ib, semi)
        pltpu.sync_copy(ones_v, acc_sh.at[ia], add=True)
        pltpu.make_async_copy(
            dst_hbm.at[pl.ds(base + j1 * _C, _C)], ib, semi).wait()

        @pl.when(p < _NPAIR - 1)
        def _():
            pltpu.async_copy(
                dst_hbm.at[pl.ds(base + (j1 + 1) * _C, _C)], ia, semi)

        pltpu.sync_copy(ones_v, acc_sh.at[ib], add=True)

        @pl.when(p < _NPAIR - 1)
        def _():
            pltpu.make_async_copy(
                dst_hbm.at[pl.ds(base + (j1 + 1) * _C, _C)], ia, semi).wait()

        return carry

    lax.fori_loop(0, _NPAIR, _pair, 0)
    pltpu.sync_copy(dst_hbm.at[pl.ds(base + (_NCH - 1) * _C, _C)], ia)
    pltpu.sync_copy(ones_v, acc_sh.at[ia], add=True)

    plsc.subcore_barrier()
    pltpu.sync_copy(acc_sh.at[pl.ds(s * _RPT, _RPT)],
                    out_hbm.at[c, pl.ds(s * _RPT, _RPT)])


_deg_kernel = functools.partial(
    pl.kernel,
    out_type=jax.ShapeDtypeStruct((_NC, _NP), jnp.float32),
    mesh=_sc_mesh,
    scratch_types=[
        pltpu.VMEM((_C,), jnp.int32),
        pltpu.VMEM((_C,), jnp.int32),
        pltpu.VMEM((_C,), jnp.float32),
        pltpu.VMEM((_RPT,), jnp.float32),
        pltpu.VMEM_SHARED((_NP,), jnp.float32),
        pltpu.SemaphoreType.DMA,
    ],
)(_deg_body)


def _msg_body(g_hbm, src_hbm, dst_hbm, out_hbm,
              sa, sb, da, db, rows_a, rows_b, zrow, acc_sh,
              semi_a, semi_b, semg_a, semg_b):
    c = lax.axis_index("c")
    s = lax.axis_index("s")
    wid = c * _NS + s
    base = wid * _EW

    def _zinit(j, carry):
        r = j // (_D // 16)
        l = j % (_D // 16)
        zrow[r, pl.ds(16 * l, 16)] = jnp.zeros((16,), jnp.float32)
        return carry

    lax.fori_loop(0, _ZB * (_D // 16), _zinit, 0)

    def _ld(sref, dref, j, sem):
        pltpu.async_copy(src_hbm.at[pl.ds(base + j * _C, _C)], sref, sem)
        pltpu.async_copy(dst_hbm.at[pl.ds(base + j * _C, _C)], dref, sem)

    def _ld_wait(sref, dref, j, sem):
        pltpu.make_async_copy(
            src_hbm.at[pl.ds(base + j * _C, _C)], sref, sem).wait()
        pltpu.make_async_copy(
            dst_hbm.at[pl.ds(base + j * _C, _C)], dref, sem).wait()

    for r in range(_RPT // _ZB):
        pltpu.sync_copy(zrow, acc_sh.at[pl.ds(s * _RPT + r * _ZB, _ZB), :])
    plsc.subcore_barrier()

    pltpu.sync_copy(src_hbm.at[pl.ds(base, _C)], sa)
    pltpu.sync_copy(dst_hbm.at[pl.ds(base, _C)], da)
    pltpu.async_copy(g_hbm.at[sa], rows_a, semg_a)

    def _pair(p, carry):
        j1 = 2 * p + 1
        _ld(sb, db, j1, semi_b)
        pltpu.make_async_copy(g_hbm.at[sa], rows_a, semg_a).wait()
        _ld_wait(sb, db, j1, semi_b)
        pltpu.async_copy(g_hbm.at[sb], rows_b, semg_b)
        pltpu.sync_copy(rows_a, acc_sh.at[da], add=True)

        @pl.when(p < _NPAIR - 1)
        def _():
            _ld(sa, da, j1 + 1, semi_a)

        pltpu.make_async_copy(g_hbm.at[sb], rows_b, semg_b).wait()

        @pl.when(p < _NPAIR - 1)
        def _():
            _ld_wait(sa, da, j1 + 1, semi_a)
            pltpu.async_copy(g_hbm.at[sa], rows_a, semg_a)

        pltpu.sync_copy(rows_b, acc_sh.at[db], add=True)
        return carry

    lax.fori_loop(0, _NPAIR, _pair, 0)
    pltpu.sync_copy(src_hbm.at[pl.ds(base + (_NCH - 1) * _C, _C)], sa)
    pltpu.sync_copy(dst_hbm.at[pl.ds(base + (_NCH - 1) * _C, _C)], da)
    pltpu.async_copy(g_hbm.at[sa], rows_a, semg_a).wait()
    pltpu.sync_copy(rows_a, acc_sh.at[da], add=True)

    plsc.subcore_barrier()
    pltpu.sync_copy(acc_sh.at[pl.ds(s * _RPT, _RPT), :],
                    out_hbm.at[c, pl.ds(s * _RPT, _RPT), :])


_msg_kernel = functools.partial(
    pl.kernel,
    out_type=jax.ShapeDtypeStruct((_NC, _NP, _D), jnp.float32),
    mesh=_sc_mesh,
    scratch_types=[
        pltpu.VMEM((_C,), jnp.int32),
        pltpu.VMEM((_C,), jnp.int32),
        pltpu.VMEM((_C,), jnp.int32),
        pltpu.VMEM((_C,), jnp.int32),
        pltpu.VMEM((_C, _D), jnp.float32),
        pltpu.VMEM((_C, _D), jnp.float32),
        pltpu.VMEM((_ZB, _D), jnp.float32),
        pltpu.VMEM_SHARED((_NP, _D), jnp.float32),
        pltpu.SemaphoreType.DMA,
        pltpu.SemaphoreType.DMA,
        pltpu.SemaphoreType.DMA,
        pltpu.SemaphoreType.DMA,
    ],
)(_msg_body)


def _g_body(x_ref, w_ref, degp_ref, g_ref):
    m = pl.program_id(0)
    deg = degp_ref[0, :] + degp_ref[1, :] + 1.0
    dinv = lax.rsqrt(deg)
    h = jnp.dot(x_ref[...], w_ref[...], preferred_element_type=jnp.float32)
    row = m * _BN + jax.lax.broadcasted_iota(jnp.int32, (_BN, 1), 0)
    g_ref[...] = jnp.where(row < _N, h * dinv[:, None], 0.0)


def _g_tc(x, W, degp):
    return pl.pallas_call(
        _g_body,
        grid=(_NP // _BN,),
        in_specs=[
            pl.BlockSpec((_BN, _D), lambda m: (m, 0)),
            pl.BlockSpec((_D, _D), lambda m: (0, 0)),
            pl.BlockSpec((_NC, _BN), lambda m: (0, m)),
        ],
        out_specs=pl.BlockSpec((_BN, _D), lambda m: (m, 0)),
        out_shape=jax.ShapeDtypeStruct((_NP, _D), jnp.float32),
    )(x, W, degp)


def _oc_body(acc_ref, g_ref, degp_ref, b_ref, oc_ref):
    m = pl.program_id(0)
    deg = degp_ref[0, :] + degp_ref[1, :] + 1.0
    dinv = lax.rsqrt(deg)
    oc = (acc_ref[0] + acc_ref[1] + g_ref[...]) * dinv[:, None] + b_ref[...]
    row = m * _BN + jax.lax.broadcasted_iota(jnp.int32, (_BN, 1), 0)
    oc_ref[...] = jnp.where(row < _N, oc, 0.0)


def _oc_tc(acc, g, degp, b2):
    return pl.pallas_call(
        _oc_body,
        grid=(_NP // _BN,),
        in_specs=[
            pl.BlockSpec((_NC, _BN, _D), lambda m: (0, m, 0)),
            pl.BlockSpec((_BN, _D), lambda m: (m, 0)),
            pl.BlockSpec((_NC, _BN), lambda m: (0, m)),
            pl.BlockSpec((1, _D), lambda m: (0, 0)),
        ],
        out_specs=pl.BlockSpec((_BN, _D), lambda m: (m, 0)),
        out_shape=jax.ShapeDtypeStruct((_NP, _D), jnp.float32),
    )(acc, g, degp, b2)


def _mm_body(origin_ref, oc_ref, out_ref, acc_ref):
    k = pl.program_id(1)
    a = origin_ref[...]
    limit = _N - k * _BK
    col = jax.lax.broadcasted_iota(jnp.int32, a.shape, 1)
    a = jnp.where(col < limit, a, 0.0)
    b = oc_ref[pl.ds(k * _BK, _BK), :]
    part = jnp.dot(a, b, preferred_element_type=jnp.float32)

    @pl.when(k == 0)
    def _():
        acc_ref[...] = jnp.zeros_like(acc_ref)

    acc_ref[...] += part

    @pl.when(k == _NKB - 1)
    def _():
        out_ref[...] = jnp.maximum(acc_ref[...], 0.0)


def _readout_matmul(origin, oc_pad):
    return pl.pallas_call(
        _mm_body,
        grid=(_N // _BM, _NKB),
        in_specs=[
            pl.BlockSpec((_BM, _BK), lambda m, k: (m, k)),
            pl.BlockSpec((_NP, _D), lambda m, k: (0, 0)),
        ],
        out_specs=pl.BlockSpec((_BM, _D), lambda m, k: (m, 0)),
        out_shape=jax.ShapeDtypeStruct((_N, _D), jnp.float32),
        scratch_shapes=[pltpu.VMEM((_BM, _D), jnp.float32)],
        compiler_params=pltpu.CompilerParams(
            dimension_semantics=("parallel", "arbitrary"),
        ),
    )(origin, oc_pad)


def kernel(shuf, origin, i, sparse, edge_index, W, b):
    src1 = edge_index[0]
    dst1 = edge_index[1]
    degp = _deg_kernel(dst1)
    g = _g_tc(shuf, W, degp)
    acc = _msg_kernel(g, src1, dst1)
    oc_pad = _oc_tc(acc, g, degp, b.reshape(1, _D))
    out = _readout_matmul(origin, oc_pad)
    return out[None]


# speedup vs baseline: 21.3501x; 1.1474x over previous
import functools

import jax
import jax.numpy as jnp
from jax import lax
from jax.experimental import pallas as pl
from jax.experimental.pallas import tpu as pltpu
from jax.experimental.pallas import tpu_sc as plsc

_N = 10000
_D = 128
_H = _D // 2
_E = 320000
_NP = 10240
_NC = 2
_NS = 16
_NW = _NC * _NS
_EW = _E // _NW
_C = 80
_NCH = _EW // _C
_NSLOT = 4
_NT = _NCH // _NSLOT
_RPT = _NP // _NS
_ZR = 128
_ZB = 32

_BM = 1000
_BK = 1024
_NKB = _NP // _BK
_BN = 1024

_sc_mesh = plsc.VectorSubcoreMesh(core_axis_name="c", subcore_axis_name="s")


def _deg_body(dst_hbm, out_hbm, i0, i1, i2, i3, ones_v, zeros_v, acc_sh,
              si0, si1, si2, si3, ss0, ss1, ss2, ss3):
    c = lax.axis_index("c")
    s = lax.axis_index("s")
    wid = c * _NS + s
    base = wid * _EW
    semi = (si0, si1, si2, si3)
    semsc = (ss0, ss1, ss2, ss3)

    for j in range(_C // 16):
        ones_v[pl.ds(16 * j, 16)] = jnp.ones((16,), jnp.float32)

    def _zinit(j, carry):
        zeros_v[pl.ds(16 * j, 16)] = jnp.zeros((16,), jnp.float32)
        return carry

    lax.fori_loop(0, _RPT // 16, _zinit, 0)
    pltpu.sync_copy(zeros_v, acc_sh.at[pl.ds(s * _RPT, _RPT)])
    plsc.subcore_barrier()

    idx_bufs = (i0, i1, i2, i3)

    def _ring(t, carry):
        for k in range(_NSLOT):
            j = t * _NSLOT + k

            @pl.when(t > 0)
            def _():
                pltpu.make_async_copy(
                    ones_v, acc_sh.at[idx_bufs[k]], semsc[k]).wait()

            pltpu.async_copy(
                dst_hbm.at[pl.ds(base + j * _C, _C)], idx_bufs[k], semi[k])
        for k in range(_NSLOT):
            j = t * _NSLOT + k
            pltpu.make_async_copy(
                dst_hbm.at[pl.ds(base + j * _C, _C)], idx_bufs[k],
                semi[k]).wait()
            pltpu.async_copy(ones_v, acc_sh.at[idx_bufs[k]], semsc[k],
                             add=True)
        return carry

    lax.fori_loop(0, _NT, _ring, 0)
    for k in range(_NSLOT):
        pltpu.make_async_copy(ones_v, acc_sh.at[idx_bufs[k]], semsc[k]).wait()
    pltpu.sync_copy(dst_hbm.at[pl.ds(base + (_NCH - 1) * _C, _C)], i0)
    pltpu.sync_copy(ones_v, acc_sh.at[i0], add=True)

    plsc.subcore_barrier()
    pltpu.sync_copy(acc_sh.at[pl.ds(s * _RPT, _RPT)],
                    out_hbm.at[c, pl.ds(s * _RPT, _RPT)])


_deg_kernel = functools.partial(
    pl.kernel,
    out_type=jax.ShapeDtypeStruct((_NC, _NP), jnp.float32),
    mesh=_sc_mesh,
    scratch_types=(
        [pltpu.VMEM((_C,), jnp.int32)] * _NSLOT
        + [pltpu.VMEM((_C,), jnp.float32),
           pltpu.VMEM((_RPT,), jnp.float32),
           pltpu.VMEM_SHARED((_NP,), jnp.float32)]
        + [pltpu.SemaphoreType.DMA] * (2 * _NSLOT)
    ),
)(_deg_body)


def _msg_body(g_hbm, src_hbm, dst_hbm, out_hbm,
              s0, s1, s2, s3, d0, d1, d2, d3, r0, r1, r2, r3, zrow, acc_sh,
              si0, si1, si2, si3, sg0, sg1, sg2, sg3, ss0, ss1, ss2, ss3):
    c = lax.axis_index("c")
    s = lax.axis_index("s")
    wid = c * _NS + s
    base = wid * _EW
    sbufs = (s0, s1, s2, s3)
    dbufs = (d0, d1, d2, d3)
    rbufs = (r0, r1, r2, r3)
    semi = (si0, si1, si2, si3)
    semg = (sg0, sg1, sg2, sg3)
    semsc = (ss0, ss1, ss2, ss3)

    def _zinit(j, carry):
        r = j // (_D // 16)
        l = j % (_D // 16)
        zrow[r, pl.ds(16 * l, 16)] = jnp.zeros((16,), jnp.float32)
        return carry

    lax.fori_loop(0, _ZB * (_D // 16), _zinit, 0)

    for r in range(_RPT // _ZB):
        pltpu.sync_copy(zrow, acc_sh.at[pl.ds(s * _RPT + r * _ZB, _ZB), :])
    plsc.subcore_barrier()

    def _ring(t, carry):
        for k in range(_NSLOT):
            j = t * _NSLOT + k

            @pl.when(t > 0)
            def _():
                pltpu.make_async_copy(
                    rbufs[k], acc_sh.at[dbufs[k]], semsc[k]).wait()

            pltpu.async_copy(
                src_hbm.at[pl.ds(base + j * _C, _C)], sbufs[k], semi[k])
            pltpu.async_copy(
                dst_hbm.at[pl.ds(base + j * _C, _C)], dbufs[k], semi[k])
        for k in range(_NSLOT):
            j = t * _NSLOT + k
            pltpu.make_async_copy(
                src_hbm.at[pl.ds(base + j * _C, _C)], sbufs[k],
                semi[k]).wait()
            pltpu.make_async_copy(
                dst_hbm.at[pl.ds(base + j * _C, _C)], dbufs[k],
                semi[k]).wait()
            pltpu.async_copy(g_hbm.at[sbufs[k]], rbufs[k], semg[k])
        for k in range(_NSLOT):
            pltpu.make_async_copy(g_hbm.at[sbufs[k]], rbufs[k],
                                  semg[k]).wait()
            pltpu.async_copy(rbufs[k], acc_sh.at[dbufs[k]], semsc[k],
                             add=True)
        return carry

    lax.fori_loop(0, _NT, _ring, 0)
    for k in range(_NSLOT):
        pltpu.make_async_copy(rbufs[k], acc_sh.at[dbufs[k]], semsc[k]).wait()
    pltpu.sync_copy(src_hbm.at[pl.ds(base + (_NCH - 1) * _C, _C)], s0)
    pltpu.sync_copy(dst_hbm.at[pl.ds(base + (_NCH - 1) * _C, _C)], d0)
    pltpu.async_copy(g_hbm.at[s0], r0, sg0).wait()
    pltpu.sync_copy(r0, acc_sh.at[d0], add=True)

    plsc.subcore_barrier()
    pltpu.sync_copy(acc_sh.at[pl.ds(s * _RPT, _RPT), :],
                    out_hbm.at[c, pl.ds(s * _RPT, _RPT), :])


_msg_kernel = functools.partial(
    pl.kernel,
    out_type=jax.ShapeDtypeStruct((_NC, _NP, _D), jnp.float32),
    mesh=_sc_mesh,
    scratch_types=(
        [pltpu.VMEM((_C,), jnp.int32)] * (2 * _NSLOT)
        + [pltpu.VMEM((_C, _D), jnp.float32)] * _NSLOT
        + [pltpu.VMEM((_ZB, _D), jnp.float32),
           pltpu.VMEM_SHARED((_NP, _D), jnp.float32)]
        + [pltpu.SemaphoreType.DMA] * (3 * _NSLOT)
    ),
)(_msg_body)


def _g_body(x_ref, w_ref, degp_ref, g_ref):
    m = pl.program_id(0)
    deg = degp_ref[0, :] + degp_ref[1, :] + 1.0
    dinv = lax.rsqrt(deg)
    h = jnp.dot(x_ref[...], w_ref[...], preferred_element_type=jnp.float32)
    row = m * _BN + jax.lax.broadcasted_iota(jnp.int32, (_BN, 1), 0)
    g_ref[...] = jnp.where(row < _N, h * dinv[:, None], 0.0)


def _g_tc(x, W, degp):
    return pl.pallas_call(
        _g_body,
        grid=(_NP // _BN,),
        in_specs=[
            pl.BlockSpec((_BN, _D), lambda m: (m, 0)),
            pl.BlockSpec((_D, _D), lambda m: (0, 0)),
            pl.BlockSpec((_NC, _BN), lambda m: (0, m)),
        ],
        out_specs=pl.BlockSpec((_BN, _D), lambda m: (m, 0)),
        out_shape=jax.ShapeDtypeStruct((_NP, _D), jnp.float32),
    )(x, W, degp)


def _oc_body(acc_ref, g_ref, degp_ref, b_ref, oc_ref):
    m = pl.program_id(0)
    deg = degp_ref[0, :] + degp_ref[1, :] + 1.0
    dinv = lax.rsqrt(deg)
    oc = (acc_ref[0] + acc_ref[1] + g_ref[...]) * dinv[:, None] + b_ref[...]
    row = m * _BN + jax.lax.broadcasted_iota(jnp.int32, (_BN, 1), 0)
    oc_ref[...] = jnp.where(row < _N, oc, 0.0)


def _oc_tc(acc, g, degp, b2):
    return pl.pallas_call(
        _oc_body,
        grid=(_NP // _BN,),
        in_specs=[
            pl.BlockSpec((_NC, _BN, _D), lambda m: (0, m, 0)),
            pl.BlockSpec((_BN, _D), lambda m: (m, 0)),
            pl.BlockSpec((_NC, _BN), lambda m: (0, m)),
            pl.BlockSpec((1, _D), lambda m: (0, 0)),
        ],
        out_specs=pl.BlockSpec((_BN, _D), lambda m: (m, 0)),
        out_shape=jax.ShapeDtypeStruct((_NP, _D), jnp.float32),
    )(acc, g, degp, b2)


def _mm_body(origin_ref, oc_ref, out_ref, acc_ref):
    k = pl.program_id(1)
    a = origin_ref[...]
    limit = _N - k * _BK
    col = jax.lax.broadcasted_iota(jnp.int32, a.shape, 1)
    a = jnp.where(col < limit, a, 0.0)
    b = oc_ref[pl.ds(k * _BK, _BK), :]
    part = jnp.dot(a, b, preferred_element_type=jnp.float32)

    @pl.when(k == 0)
    def _():
        acc_ref[...] = jnp.zeros_like(acc_ref)

    acc_ref[...] += part

    @pl.when(k == _NKB - 1)
    def _():
        out_ref[...] = jnp.maximum(acc_ref[...], 0.0)


def _readout_matmul(origin, oc_pad):
    return pl.pallas_call(
        _mm_body,
        grid=(_N // _BM, _NKB),
        in_specs=[
            pl.BlockSpec((_BM, _BK), lambda m, k: (m, k)),
            pl.BlockSpec((_NP, _D), lambda m, k: (0, 0)),
        ],
        out_specs=pl.BlockSpec((_BM, _D), lambda m, k: (m, 0)),
        out_shape=jax.ShapeDtypeStruct((_N, _D), jnp.float32),
        scratch_shapes=[pltpu.VMEM((_BM, _D), jnp.float32)],
        compiler_params=pltpu.CompilerParams(
            dimension_semantics=("parallel", "arbitrary"),
        ),
    )(origin, oc_pad)


def kernel(shuf, origin, i, sparse, edge_index, W, b):
    src1 = edge_index[0]
    dst1 = edge_index[1]
    degp = _deg_kernel(dst1)
    g = _g_tc(shuf, W, degp)
    acc = _msg_kernel(g, src1, dst1)
    oc_pad = _oc_tc(acc, g, degp, b.reshape(1, _D))
    out = _readout_matmul(origin, oc_pad)
    return out[None]


# speedup vs baseline: 24.0680x; 1.1273x over previous
import functools

import jax
import jax.numpy as jnp
from jax import lax
from jax.experimental import pallas as pl
from jax.experimental.pallas import tpu as pltpu
from jax.experimental.pallas import tpu_sc as plsc

_N = 10000
_D = 128
_H = _D // 2
_E = 320000
_NP = 10240
_NC = 2
_NS = 16
_NW = _NC * _NS
_EW = _E // _NW
_C = 80
_NCH = _EW // _C
_NSLOT = 4
_NT = _NCH // _NSLOT
_RPT = _NP // _NS
_ZR = 128
_ZB = 32

_BM = 2000
_BK = 1024
_NKB = _NP // _BK
_BN = 1024

_sc_mesh = plsc.VectorSubcoreMesh(core_axis_name="c", subcore_axis_name="s")


def _deg_body(edge_hbm, out_hbm, i0, i1, i2, i3, ones_v, zeros_v, acc_sh,
              si0, si1, si2, si3, ss0, ss1, ss2, ss3):
    c = lax.axis_index("c")
    s = lax.axis_index("s")
    wid = c * _NS + s
    base = _E + wid * _EW
    dst_hbm = edge_hbm
    semi = (si0, si1, si2, si3)
    semsc = (ss0, ss1, ss2, ss3)

    for j in range(_C // 16):
        ones_v[pl.ds(16 * j, 16)] = jnp.ones((16,), jnp.float32)

    def _zinit(j, carry):
        zeros_v[pl.ds(16 * j, 16)] = jnp.zeros((16,), jnp.float32)
        return carry

    lax.fori_loop(0, _RPT // 16, _zinit, 0)
    pltpu.sync_copy(zeros_v, acc_sh.at[pl.ds(s * _RPT, _RPT)])
    plsc.subcore_barrier()

    idx_bufs = (i0, i1, i2, i3)

    def _ring(t, carry):
        for k in range(_NSLOT):
            j = t * _NSLOT + k

            @pl.when(t > 0)
            def _():
                pltpu.make_async_copy(
                    ones_v, acc_sh.at[idx_bufs[k]], semsc[k]).wait()

            pltpu.async_copy(
                dst_hbm.at[pl.ds(base + j * _C, _C)], idx_bufs[k], semi[k])
        for k in range(_NSLOT):
            j = t * _NSLOT + k
            pltpu.make_async_copy(
                dst_hbm.at[pl.ds(base + j * _C, _C)], idx_bufs[k],
                semi[k]).wait()
            pltpu.async_copy(ones_v, acc_sh.at[idx_bufs[k]], semsc[k],
                             add=True)
        return carry

    lax.fori_loop(0, _NT, _ring, 0)
    for k in range(_NSLOT):
        pltpu.make_async_copy(ones_v, acc_sh.at[idx_bufs[k]], semsc[k]).wait()
    pltpu.sync_copy(dst_hbm.at[pl.ds(base + (_NCH - 1) * _C, _C)], i0)
    pltpu.sync_copy(ones_v, acc_sh.at[i0], add=True)

    plsc.subcore_barrier()
    pltpu.sync_copy(acc_sh.at[pl.ds(s * _RPT, _RPT)],
                    out_hbm.at[c, pl.ds(s * _RPT, _RPT)])


_deg_kernel = functools.partial(
    pl.kernel,
    out_type=jax.ShapeDtypeStruct((_NC, _NP), jnp.float32),
    mesh=_sc_mesh,
    scratch_types=(
        [pltpu.VMEM((_C,), jnp.int32)] * _NSLOT
        + [pltpu.VMEM((_C,), jnp.float32),
           pltpu.VMEM((_RPT,), jnp.float32),
           pltpu.VMEM_SHARED((_NP,), jnp.float32)]
        + [pltpu.SemaphoreType.DMA] * (2 * _NSLOT)
    ),
)(_deg_body)


def _msg_body(g_hbm, edge_hbm, out_hbm,
              s0, s1, s2, s3, d0, d1, d2, d3, r0, r1, r2, r3, zrow, acc_sh,
              si0, si1, si2, si3, sg0, sg1, sg2, sg3, ss0, ss1, ss2, ss3):
    c = lax.axis_index("c")
    s = lax.axis_index("s")
    wid = c * _NS + s
    base = wid * _EW
    src_hbm = edge_hbm
    dst_hbm = edge_hbm
    dbase = _E + base
    sbufs = (s0, s1, s2, s3)
    dbufs = (d0, d1, d2, d3)
    rbufs = (r0, r1, r2, r3)
    semi = (si0, si1, si2, si3)
    semg = (sg0, sg1, sg2, sg3)
    semsc = (ss0, ss1, ss2, ss3)

    def _zinit(j, carry):
        r = j // (_D // 16)
        l = j % (_D // 16)
        zrow[r, pl.ds(16 * l, 16)] = jnp.zeros((16,), jnp.float32)
        return carry

    lax.fori_loop(0, _ZB * (_D // 16), _zinit, 0)

    for r in range(_RPT // _ZB):
        pltpu.sync_copy(zrow, acc_sh.at[pl.ds(s * _RPT + r * _ZB, _ZB), :])
    plsc.subcore_barrier()

    def _ring(t, carry):
        for k in range(_NSLOT):
            j = t * _NSLOT + k

            @pl.when(t > 0)
            def _():
                pltpu.make_async_copy(
                    rbufs[k], acc_sh.at[dbufs[k]], semsc[k]).wait()

            pltpu.async_copy(
                src_hbm.at[pl.ds(base + j * _C, _C)], sbufs[k], semi[k])
            pltpu.async_copy(
                dst_hbm.at[pl.ds(dbase + j * _C, _C)], dbufs[k], semi[k])
        for k in range(_NSLOT):
            j = t * _NSLOT + k
            pltpu.make_async_copy(
                src_hbm.at[pl.ds(base + j * _C, _C)], sbufs[k],
                semi[k]).wait()
            pltpu.make_async_copy(
                dst_hbm.at[pl.ds(dbase + j * _C, _C)], dbufs[k],
                semi[k]).wait()
            pltpu.async_copy(g_hbm.at[sbufs[k]], rbufs[k], semg[k])
        for k in range(_NSLOT):
            pltpu.make_async_copy(g_hbm.at[sbufs[k]], rbufs[k],
                                  semg[k]).wait()
            pltpu.async_copy(rbufs[k], acc_sh.at[dbufs[k]], semsc[k],
                             add=True)
        return carry

    lax.fori_loop(0, _NT, _ring, 0)
    for k in range(_NSLOT):
        pltpu.make_async_copy(rbufs[k], acc_sh.at[dbufs[k]], semsc[k]).wait()
    pltpu.sync_copy(src_hbm.at[pl.ds(base + (_NCH - 1) * _C, _C)], s0)
    pltpu.sync_copy(dst_hbm.at[pl.ds(dbase + (_NCH - 1) * _C, _C)], d0)
    pltpu.async_copy(g_hbm.at[s0], r0, sg0).wait()
    pltpu.sync_copy(r0, acc_sh.at[d0], add=True)

    plsc.subcore_barrier()
    pltpu.sync_copy(acc_sh.at[pl.ds(s * _RPT, _RPT), :],
                    out_hbm.at[c, pl.ds(s * _RPT, _RPT), :])


_msg_kernel = functools.partial(
    pl.kernel,
    out_type=jax.ShapeDtypeStruct((_NC, _NP, _D), jnp.float32),
    mesh=_sc_mesh,
    scratch_types=(
        [pltpu.VMEM((_C,), jnp.int32)] * (2 * _NSLOT)
        + [pltpu.VMEM((_C, _D), jnp.float32)] * _NSLOT
        + [pltpu.VMEM((_ZB, _D), jnp.float32),
           pltpu.VMEM_SHARED((_NP, _D), jnp.float32)]
        + [pltpu.SemaphoreType.DMA] * (3 * _NSLOT)
    ),
)(_msg_body)


def _g_body(x_ref, w_ref, degp_ref, g_ref):
    m = pl.program_id(0)
    deg = degp_ref[0, :] + degp_ref[1, :] + 1.0
    dinv = lax.rsqrt(deg)
    h = jnp.dot(x_ref[...], w_ref[...], preferred_element_type=jnp.float32)
    row = m * _BN + jax.lax.broadcasted_iota(jnp.int32, (_BN, 1), 0)
    g_ref[...] = jnp.where(row < _N, h * dinv[:, None], 0.0)


def _g_tc(x, W, degp):
    return pl.pallas_call(
        _g_body,
        grid=(_NP // _BN,),
        in_specs=[
            pl.BlockSpec((_BN, _D), lambda m: (m, 0)),
            pl.BlockSpec((_D, _D), lambda m: (0, 0)),
            pl.BlockSpec((_NC, _BN), lambda m: (0, m)),
        ],
        out_specs=pl.BlockSpec((_BN, _D), lambda m: (m, 0)),
        out_shape=jax.ShapeDtypeStruct((_NP, _D), jnp.float32),
    )(x, W, degp)


def _oc_body(acc_ref, g_ref, degp_ref, b_ref, oc_ref):
    m = pl.program_id(0)
    deg = degp_ref[0, :] + degp_ref[1, :] + 1.0
    dinv = lax.rsqrt(deg)
    oc = (acc_ref[0] + acc_ref[1] + g_ref[...]) * dinv[:, None] + b_ref[...]
    row = m * _BN + jax.lax.broadcasted_iota(jnp.int32, (_BN, 1), 0)
    oc_ref[...] = jnp.where(row < _N, oc, 0.0)


def _oc_tc(acc, g, degp, b2):
    return pl.pallas_call(
        _oc_body,
        grid=(_NP // _BN,),
        in_specs=[
            pl.BlockSpec((_NC, _BN, _D), lambda m: (0, m, 0)),
            pl.BlockSpec((_BN, _D), lambda m: (m, 0)),
            pl.BlockSpec((_NC, _BN), lambda m: (0, m)),
            pl.BlockSpec((1, _D), lambda m: (0, 0)),
        ],
        out_specs=pl.BlockSpec((_BN, _D), lambda m: (m, 0)),
        out_shape=jax.ShapeDtypeStruct((_NP, _D), jnp.float32),
    )(acc, g, degp, b2)


def _mm_body(origin_ref, oc_ref, out_ref, acc_ref):
    k = pl.program_id(1)
    a = origin_ref[...]
    limit = _N - k * _BK
    col = jax.lax.broadcasted_iota(jnp.int32, a.shape, 1)
    a = jnp.where(col < limit, a, 0.0)
    b = oc_ref[pl.ds(k * _BK, _BK), :]
    part = jnp.dot(a, b, preferred_element_type=jnp.float32)

    @pl.when(k == 0)
    def _():
        acc_ref[...] = jnp.zeros_like(acc_ref)

    acc_ref[...] += part

    @pl.when(k == _NKB - 1)
    def _():
        out_ref[...] = jnp.maximum(acc_ref[...], 0.0)


def _readout_matmul(origin, oc_pad):
    return pl.pallas_call(
        _mm_body,
        grid=(_N // _BM, _NKB),
        in_specs=[
            pl.BlockSpec((_BM, _BK), lambda m, k: (m, k)),
            pl.BlockSpec((_NP, _D), lambda m, k: (0, 0)),
        ],
        out_specs=pl.BlockSpec((_BM, _D), lambda m, k: (m, 0)),
        out_shape=jax.ShapeDtypeStruct((_N, _D), jnp.float32),
        scratch_shapes=[pltpu.VMEM((_BM, _D), jnp.float32)],
        compiler_params=pltpu.CompilerParams(
            dimension_semantics=("parallel", "arbitrary"),
        ),
    )(origin, oc_pad)


def kernel(shuf, origin, i, sparse, edge_index, W, b):
    edge_flat = jnp.ravel(edge_index)
    degp = _deg_kernel(edge_flat)
    g = _g_tc(shuf, W, degp)
    acc = _msg_kernel(g, edge_flat)
    oc_pad = _oc_tc(acc, g, degp, b.reshape(1, _D))
    out = _readout_matmul(origin, oc_pad)
    return out[None]


# speedup vs baseline: 25.4570x; 1.0577x over previous
import functools

import jax
import jax.numpy as jnp
from jax import lax
from jax.experimental import pallas as pl
from jax.experimental.pallas import tpu as pltpu
from jax.experimental.pallas import tpu_sc as plsc

_N = 10000
_D = 128
_H = _D // 2
_E = 320000
_NP = 10240
_NC = 2
_NS = 16
_NW = _NC * _NS
_EW = _E // _NW
_C = 80
_NCH = _EW // _C
_NSLOT = 4
_NT = _NCH // _NSLOT
_RPT = _NP // _NS
_ZR = 128
_ZB = 32

_BM = 2000
_BK = 1024
_NKB = _NP // _BK
_BN = 1024

_sc_mesh = plsc.VectorSubcoreMesh(core_axis_name="c", subcore_axis_name="s")


def _deg_body(edge_hbm, out_hbm, i0, i1, i2, i3, ones_v, zeros_v, acc_sh,
              si0, si1, si2, si3, ss0, ss1, ss2, ss3):
    c = lax.axis_index("c")
    s = lax.axis_index("s")
    wid = c * _NS + s
    base = _E + wid * _EW
    dst_hbm = edge_hbm
    semi = (si0, si1, si2, si3)
    semsc = (ss0, ss1, ss2, ss3)

    for j in range(_C // 16):
        ones_v[pl.ds(16 * j, 16)] = jnp.ones((16,), jnp.float32)

    def _zinit(j, carry):
        zeros_v[pl.ds(16 * j, 16)] = jnp.zeros((16,), jnp.float32)
        return carry

    lax.fori_loop(0, _RPT // 16, _zinit, 0)
    pltpu.sync_copy(zeros_v, acc_sh.at[pl.ds(s * _RPT, _RPT)])
    plsc.subcore_barrier()

    idx_bufs = (i0, i1, i2, i3)

    def _ring(t, carry):
        for k in range(_NSLOT):
            j = t * _NSLOT + k

            @pl.when(t > 0)
            def _():
                pltpu.make_async_copy(
                    ones_v, acc_sh.at[idx_bufs[k]], semsc[k]).wait()

            pltpu.async_copy(
                dst_hbm.at[pl.ds(base + j * _C, _C)], idx_bufs[k], semi[k])
        for k in range(_NSLOT):
            j = t * _NSLOT + k
            pltpu.make_async_copy(
                dst_hbm.at[pl.ds(base + j * _C, _C)], idx_bufs[k],
                semi[k]).wait()
            pltpu.async_copy(ones_v, acc_sh.at[idx_bufs[k]], semsc[k],
                             add=True)
        return carry

    lax.fori_loop(0, _NT, _ring, 0)
    for k in range(_NSLOT):
        pltpu.make_async_copy(ones_v, acc_sh.at[idx_bufs[k]], semsc[k]).wait()
    pltpu.sync_copy(dst_hbm.at[pl.ds(base + (_NCH - 1) * _C, _C)], i0)
    pltpu.sync_copy(ones_v, acc_sh.at[i0], add=True)

    plsc.subcore_barrier()
    pltpu.sync_copy(acc_sh.at[pl.ds(s * _RPT, _RPT)],
                    out_hbm.at[c, pl.ds(s * _RPT, _RPT)])


_deg_kernel = functools.partial(
    pl.kernel,
    out_type=jax.ShapeDtypeStruct((_NC, _NP), jnp.float32),
    mesh=_sc_mesh,
    scratch_types=(
        [pltpu.VMEM((_C,), jnp.int32)] * _NSLOT
        + [pltpu.VMEM((_C,), jnp.float32),
           pltpu.VMEM((_RPT,), jnp.float32),
           pltpu.VMEM_SHARED((_NP,), jnp.float32)]
        + [pltpu.SemaphoreType.DMA] * (2 * _NSLOT)
    ),
)(_deg_body)


_NB2 = _NCH // (2 * _NSLOT)
_NTAIL = _NCH - _NB2 * 2 * _NSLOT


def _msg_body(g_hbm, edge_hbm, out_hbm, *refs):
    c = lax.axis_index("c")
    s = lax.axis_index("s")
    wid = c * _NS + s
    base = wid * _EW
    src_hbm = edge_hbm
    dst_hbm = edge_hbm
    dbase = _E + base
    sA = refs[0:4]
    dA = refs[4:8]
    sB = refs[8:12]
    dB = refs[12:16]
    rbufs = refs[16:20]
    zrow = refs[20]
    acc_sh = refs[21]
    semi_a = refs[22:26]
    semi_b = refs[26:30]
    semg = refs[30:34]
    semsc = refs[34:38]

    def _zinit(j, carry):
        r = j // (_D // 16)
        l = j % (_D // 16)
        zrow[r, pl.ds(16 * l, 16)] = jnp.zeros((16,), jnp.float32)
        return carry

    lax.fori_loop(0, _ZB * (_D // 16), _zinit, 0)

    for r in range(_RPT // _ZB):
        pltpu.sync_copy(zrow, acc_sh.at[pl.ds(s * _RPT + r * _ZB, _ZB), :])
    plsc.subcore_barrier()

    def _ld(j, sref, dref, sem):
        pltpu.async_copy(src_hbm.at[pl.ds(base + j * _C, _C)], sref, sem)
        pltpu.async_copy(dst_hbm.at[pl.ds(dbase + j * _C, _C)], dref, sem)

    def _ld_wait(j, sref, dref, sem):
        pltpu.make_async_copy(
            src_hbm.at[pl.ds(base + j * _C, _C)], sref, sem).wait()
        pltpu.make_async_copy(
            dst_hbm.at[pl.ds(dbase + j * _C, _C)], dref, sem).wait()

    for k in range(_NSLOT):
        _ld(k, sA[k], dA[k], semi_a[k])

    def _body(t, carry):
        b0 = 2 * _NSLOT * t
        for k in range(_NSLOT):
            _ld_wait(b0 + k, sA[k], dA[k], semi_a[k])

            @pl.when(t > 0)
            def _():
                pltpu.make_async_copy(
                    rbufs[k], acc_sh.at[dB[k]], semsc[k]).wait()

            pltpu.async_copy(g_hbm.at[sA[k]], rbufs[k], semg[k])
        for k in range(_NSLOT):
            _ld(b0 + _NSLOT + k, sB[k], dB[k], semi_b[k])
        for k in range(_NSLOT):
            pltpu.make_async_copy(g_hbm.at[sA[k]], rbufs[k], semg[k]).wait()
            pltpu.async_copy(rbufs[k], acc_sh.at[dA[k]], semsc[k], add=True)
        for k in range(_NSLOT):
            _ld_wait(b0 + _NSLOT + k, sB[k], dB[k], semi_b[k])
            pltpu.make_async_copy(rbufs[k], acc_sh.at[dA[k]], semsc[k]).wait()
            pltpu.async_copy(g_hbm.at[sB[k]], rbufs[k], semg[k])

        @pl.when(t < _NB2 - 1)
        def _():
            for k in range(_NSLOT):
                _ld(b0 + 2 * _NSLOT + k, sA[k], dA[k], semi_a[k])

        for k in range(_NSLOT):
            pltpu.make_async_copy(g_hbm.at[sB[k]], rbufs[k], semg[k]).wait()
            pltpu.async_copy(rbufs[k], acc_sh.at[dB[k]], semsc[k], add=True)
        return carry

    lax.fori_loop(0, _NB2, _body, 0)
    for k in range(_NSLOT):
        pltpu.make_async_copy(rbufs[k], acc_sh.at[dB[k]], semsc[k]).wait()
    for jj in range(_NB2 * 2 * _NSLOT, _NCH):
        pltpu.sync_copy(src_hbm.at[pl.ds(base + jj * _C, _C)], sA[0])
        pltpu.sync_copy(dst_hbm.at[pl.ds(dbase + jj * _C, _C)], dA[0])
        pltpu.async_copy(g_hbm.at[sA[0]], rbufs[0], semg[0]).wait()
        pltpu.sync_copy(rbufs[0], acc_sh.at[dA[0]], add=True)

    plsc.subcore_barrier()
    pltpu.sync_copy(acc_sh.at[pl.ds(s * _RPT, _RPT), :],
                    out_hbm.at[c, pl.ds(s * _RPT, _RPT), :])


_msg_kernel = functools.partial(
    pl.kernel,
    out_type=jax.ShapeDtypeStruct((_NC, _NP, _D), jnp.float32),
    mesh=_sc_mesh,
    scratch_types=(
        [pltpu.VMEM((_C,), jnp.int32)] * (4 * _NSLOT)
        + [pltpu.VMEM((_C, _D), jnp.float32)] * _NSLOT
        + [pltpu.VMEM((_ZB, _D), jnp.float32),
           pltpu.VMEM_SHARED((_NP, _D), jnp.float32)]
        + [pltpu.SemaphoreType.DMA] * (4 * _NSLOT)
    ),
)(_msg_body)


def _g_body(x_ref, w_ref, degp_ref, g_ref):
    m = pl.program_id(0)
    deg = degp_ref[0, :] + degp_ref[1, :] + 1.0
    dinv = lax.rsqrt(deg)
    h = jnp.dot(x_ref[...], w_ref[...], preferred_element_type=jnp.float32)
    row = m * _BN + jax.lax.broadcasted_iota(jnp.int32, (_BN, 1), 0)
    g_ref[...] = jnp.where(row < _N, h * dinv[:, None], 0.0)


def _g_tc(x, W, degp):
    return pl.pallas_call(
        _g_body,
        grid=(_NP // _BN,),
        in_specs=[
            pl.BlockSpec((_BN, _D), lambda m: (m, 0)),
            pl.BlockSpec((_D, _D), lambda m: (0, 0)),
            pl.BlockSpec((_NC, _BN), lambda m: (0, m)),
        ],
        out_specs=pl.BlockSpec((_BN, _D), lambda m: (m, 0)),
        out_shape=jax.ShapeDtypeStruct((_NP, _D), jnp.float32),
    )(x, W, degp)


def _oc_body(acc_ref, g_ref, degp_ref, b_ref, oc_ref):
    m = pl.program_id(0)
    deg = degp_ref[0, :] + degp_ref[1, :] + 1.0
    dinv = lax.rsqrt(deg)
    oc = (acc_ref[0] + acc_ref[1] + g_ref[...]) * dinv[:, None] + b_ref[...]
    row = m * _BN + jax.lax.broadcasted_iota(jnp.int32, (_BN, 1), 0)
    oc_ref[...] = jnp.where(row < _N, oc, 0.0)


def _oc_tc(acc, g, degp, b2):
    return pl.pallas_call(
        _oc_body,
        grid=(_NP // _BN,),
        in_specs=[
            pl.BlockSpec((_NC, _BN, _D), lambda m: (0, m, 0)),
            pl.BlockSpec((_BN, _D), lambda m: (m, 0)),
            pl.BlockSpec((_NC, _BN), lambda m: (0, m)),
            pl.BlockSpec((1, _D), lambda m: (0, 0)),
        ],
        out_specs=pl.BlockSpec((_BN, _D), lambda m: (m, 0)),
        out_shape=jax.ShapeDtypeStruct((_NP, _D), jnp.float32),
    )(acc, g, degp, b2)


def _mm_body(origin_ref, oc_ref, out_ref, acc_ref):
    k = pl.program_id(1)
    a = origin_ref[...]
    limit = _N - k * _BK
    col = jax.lax.broadcasted_iota(jnp.int32, a.shape, 1)
    a = jnp.where(col < limit, a, 0.0)
    b = oc_ref[pl.ds(k * _BK, _BK), :]
    part = jnp.dot(a, b, preferred_element_type=jnp.float32)

    @pl.when(k == 0)
    def _():
        acc_ref[...] = jnp.zeros_like(acc_ref)

    acc_ref[...] += part

    @pl.when(k == _NKB - 1)
    def _():
        out_ref[...] = jnp.maximum(acc_ref[...], 0.0)


def _readout_matmul(origin, oc_pad):
    return pl.pallas_call(
        _mm_body,
        grid=(_N // _BM, _NKB),
        in_specs=[
            pl.BlockSpec((_BM, _BK), lambda m, k: (m, k)),
            pl.BlockSpec((_NP, _D), lambda m, k: (0, 0)),
        ],
        out_specs=pl.BlockSpec((_BM, _D), lambda m, k: (m, 0)),
        out_shape=jax.ShapeDtypeStruct((_N, _D), jnp.float32),
        scratch_shapes=[pltpu.VMEM((_BM, _D), jnp.float32)],
        compiler_params=pltpu.CompilerParams(
            dimension_semantics=("parallel", "arbitrary"),
        ),
    )(origin, oc_pad)


def kernel(shuf, origin, i, sparse, edge_index, W, b):
    edge_flat = jnp.ravel(edge_index)
    degp = _deg_kernel(edge_flat)
    g = _g_tc(shuf, W, degp)
    acc = _msg_kernel(g, edge_flat)
    oc_pad = _oc_tc(acc, g, degp, b.reshape(1, _D))
    out = _readout_matmul(origin, oc_pad)
    return out[None]


# speedup vs baseline: 25.7996x; 1.0135x over previous
import functools

import jax
import jax.numpy as jnp
from jax import lax
from jax.experimental import pallas as pl
from jax.experimental.pallas import tpu as pltpu
from jax.experimental.pallas import tpu_sc as plsc

_N = 10000
_D = 128
_H = _D // 2
_E = 320000
_NP = 10240
_NC = 2
_NS = 16
_NW = _NC * _NS
_EW = _E // _NW
_C = 80
_NCH = _EW // _C
_NSLOT = 4
_NT = _NCH // _NSLOT
_RPT = _NP // _NS
_ZR = 128
_ZB = 32

_BM = 2000
_BK = 2048
_NKB = _NP // _BK
_BN = 1024

_sc_mesh = plsc.VectorSubcoreMesh(core_axis_name="c", subcore_axis_name="s")


def _deg_body(edge_hbm, out_hbm, *refs):
    c = lax.axis_index("c")
    s = lax.axis_index("s")
    wid = c * _NS + s
    base = _E + wid * _EW
    dst_hbm = edge_hbm
    iA = refs[0:4]
    iB = refs[4:8]
    ones_v = refs[8]
    zeros_v = refs[9]
    acc_sh = refs[10]
    semiA = refs[11:15]
    semiB = refs[15:19]
    ssA = refs[19:23]
    ssB = refs[23:27]

    for j in range(_C // 16):
        ones_v[pl.ds(16 * j, 16)] = jnp.ones((16,), jnp.float32)

    def _zinit(j, carry):
        zeros_v[pl.ds(16 * j, 16)] = jnp.zeros((16,), jnp.float32)
        return carry

    lax.fori_loop(0, _RPT // 16, _zinit, 0)
    pltpu.sync_copy(zeros_v, acc_sh.at[pl.ds(s * _RPT, _RPT)])
    plsc.subcore_barrier()

    def _ld(j, ref, sem):
        pltpu.async_copy(dst_hbm.at[pl.ds(base + j * _C, _C)], ref, sem)

    def _ld_wait(j, ref, sem):
        pltpu.make_async_copy(
            dst_hbm.at[pl.ds(base + j * _C, _C)], ref, sem).wait()

    for k in range(_NSLOT):
        _ld(k, iA[k], semiA[k])

    def _body(t, carry):
        b0 = 2 * _NSLOT * t
        for k in range(_NSLOT):
            _ld_wait(b0 + k, iA[k], semiA[k])

            @pl.when(t > 0)
            def _():
                pltpu.make_async_copy(
                    ones_v, acc_sh.at[iB[k]], ssB[k]).wait()

            pltpu.async_copy(ones_v, acc_sh.at[iA[k]], ssA[k], add=True)
        for k in range(_NSLOT):
            _ld(b0 + _NSLOT + k, iB[k], semiB[k])
        for k in range(_NSLOT):
            _ld_wait(b0 + _NSLOT + k, iB[k], semiB[k])
            pltpu.make_async_copy(ones_v, acc_sh.at[iA[k]], ssA[k]).wait()
            pltpu.async_copy(ones_v, acc_sh.at[iB[k]], ssB[k], add=True)

        @pl.when(t < _NB2 - 1)
        def _():
            for k in range(_NSLOT):
                _ld(b0 + 2 * _NSLOT + k, iA[k], semiA[k])

        return carry

    lax.fori_loop(0, _NB2, _body, 0)
    for k in range(_NSLOT):
        pltpu.make_async_copy(ones_v, acc_sh.at[iB[k]], ssB[k]).wait()
    for jj in range(_NB2 * 2 * _NSLOT, _NCH):
        pltpu.sync_copy(dst_hbm.at[pl.ds(base + jj * _C, _C)], iA[0])
        pltpu.sync_copy(ones_v, acc_sh.at[iA[0]], add=True)

    plsc.subcore_barrier()
    pltpu.sync_copy(acc_sh.at[pl.ds(s * _RPT, _RPT)],
                    out_hbm.at[c, pl.ds(s * _RPT, _RPT)])


_deg_kernel = functools.partial(
    pl.kernel,
    out_type=jax.ShapeDtypeStruct((_NC, _NP), jnp.float32),
    mesh=_sc_mesh,
    scratch_types=(
        [pltpu.VMEM((_C,), jnp.int32)] * (2 * _NSLOT)
        + [pltpu.VMEM((_C,), jnp.float32),
           pltpu.VMEM((_RPT,), jnp.float32),
           pltpu.VMEM_SHARED((_NP,), jnp.float32)]
        + [pltpu.SemaphoreType.DMA] * (4 * _NSLOT)
    ),
)(_deg_body)


_NB2 = _NCH // (2 * _NSLOT)
_NTAIL = _NCH - _NB2 * 2 * _NSLOT


def _msg_body(g_hbm, edge_hbm, out_hbm, *refs):
    c = lax.axis_index("c")
    s = lax.axis_index("s")
    wid = c * _NS + s
    base = wid * _EW
    src_hbm = edge_hbm
    dst_hbm = edge_hbm
    dbase = _E + base
    sA = refs[0:4]
    dA = refs[4:8]
    sB = refs[8:12]
    dB = refs[12:16]
    rbufs = refs[16:20]
    zrow = refs[20]
    acc_sh = refs[21]
    semi_a = refs[22:26]
    semi_b = refs[26:30]
    semg = refs[30:34]
    semsc = refs[34:38]

    def _zinit(j, carry):
        r = j // (_D // 16)
        l = j % (_D // 16)
        zrow[r, pl.ds(16 * l, 16)] = jnp.zeros((16,), jnp.float32)
        return carry

    lax.fori_loop(0, _ZB * (_D // 16), _zinit, 0)

    for r in range(_RPT // _ZB):
        pltpu.sync_copy(zrow, acc_sh.at[pl.ds(s * _RPT + r * _ZB, _ZB), :])
    plsc.subcore_barrier()

    def _ld(j, sref, dref, sem):
        pltpu.async_copy(src_hbm.at[pl.ds(base + j * _C, _C)], sref, sem)
        pltpu.async_copy(dst_hbm.at[pl.ds(dbase + j * _C, _C)], dref, sem)

    def _ld_wait(j, sref, dref, sem):
        pltpu.make_async_copy(
            src_hbm.at[pl.ds(base + j * _C, _C)], sref, sem).wait()
        pltpu.make_async_copy(
            dst_hbm.at[pl.ds(dbase + j * _C, _C)], dref, sem).wait()

    for k in range(_NSLOT):
        _ld(k, sA[k], dA[k], semi_a[k])

    def _body(t, carry):
        b0 = 2 * _NSLOT * t
        for k in range(_NSLOT):
            _ld_wait(b0 + k, sA[k], dA[k], semi_a[k])

            @pl.when(t > 0)
            def _():
                pltpu.make_async_copy(
                    rbufs[k], acc_sh.at[dB[k]], semsc[k]).wait()

            pltpu.async_copy(g_hbm.at[sA[k]], rbufs[k], semg[k])
        for k in range(_NSLOT):
            _ld(b0 + _NSLOT + k, sB[k], dB[k], semi_b[k])
        for k in range(_NSLOT):
            pltpu.make_async_copy(g_hbm.at[sA[k]], rbufs[k], semg[k]).wait()
            pltpu.async_copy(rbufs[k], acc_sh.at[dA[k]], semsc[k], add=True)
        for k in range(_NSLOT):
            _ld_wait(b0 + _NSLOT + k, sB[k], dB[k], semi_b[k])
            pltpu.make_async_copy(rbufs[k], acc_sh.at[dA[k]], semsc[k]).wait()
            pltpu.async_copy(g_hbm.at[sB[k]], rbufs[k], semg[k])

        @pl.when(t < _NB2 - 1)
        def _():
            for k in range(_NSLOT):
                _ld(b0 + 2 * _NSLOT + k, sA[k], dA[k], semi_a[k])

        for k in range(_NSLOT):
            pltpu.make_async_copy(g_hbm.at[sB[k]], rbufs[k], semg[k]).wait()
            pltpu.async_copy(rbufs[k], acc_sh.at[dB[k]], semsc[k], add=True)
        return carry

    lax.fori_loop(0, _NB2, _body, 0)
    for k in range(_NSLOT):
        pltpu.make_async_copy(rbufs[k], acc_sh.at[dB[k]], semsc[k]).wait()
    for jj in range(_NB2 * 2 * _NSLOT, _NCH):
        pltpu.sync_copy(src_hbm.at[pl.ds(base + jj * _C, _C)], sA[0])
        pltpu.sync_copy(dst_hbm.at[pl.ds(dbase + jj * _C, _C)], dA[0])
        pltpu.async_copy(g_hbm.at[sA[0]], rbufs[0], semg[0]).wait()
        pltpu.sync_copy(rbufs[0], acc_sh.at[dA[0]], add=True)

    plsc.subcore_barrier()
    pltpu.sync_copy(acc_sh.at[pl.ds(s * _RPT, _RPT), :],
                    out_hbm.at[c, pl.ds(s * _RPT, _RPT), :])


_msg_kernel = functools.partial(
    pl.kernel,
    out_type=jax.ShapeDtypeStruct((_NC, _NP, _D), jnp.float32),
    mesh=_sc_mesh,
    scratch_types=(
        [pltpu.VMEM((_C,), jnp.int32)] * (4 * _NSLOT)
        + [pltpu.VMEM((_C, _D), jnp.float32)] * _NSLOT
        + [pltpu.VMEM((_ZB, _D), jnp.float32),
           pltpu.VMEM_SHARED((_NP, _D), jnp.float32)]
        + [pltpu.SemaphoreType.DMA] * (4 * _NSLOT)
    ),
)(_msg_body)


def _g_body(x_ref, w_ref, degp_ref, g_ref):
    m = pl.program_id(0)
    deg = degp_ref[0, :] + degp_ref[1, :] + 1.0
    dinv = lax.rsqrt(deg)
    h = jnp.dot(x_ref[...], w_ref[...], preferred_element_type=jnp.float32)
    row = m * _BN + jax.lax.broadcasted_iota(jnp.int32, (_BN, 1), 0)
    g_ref[...] = jnp.where(row < _N, h * dinv[:, None], 0.0)


def _g_tc(x, W, degp):
    return pl.pallas_call(
        _g_body,
        grid=(_NP // _BN,),
        in_specs=[
            pl.BlockSpec((_BN, _D), lambda m: (m, 0)),
            pl.BlockSpec((_D, _D), lambda m: (0, 0)),
            pl.BlockSpec((_NC, _BN), lambda m: (0, m)),
        ],
        out_specs=pl.BlockSpec((_BN, _D), lambda m: (m, 0)),
        out_shape=jax.ShapeDtypeStruct((_NP, _D), jnp.float32),
    )(x, W, degp)


def _oc_body(acc_ref, g_ref, degp_ref, b_ref, oc_ref):
    m = pl.program_id(0)
    deg = degp_ref[0, :] + degp_ref[1, :] + 1.0
    dinv = lax.rsqrt(deg)
    oc = (acc_ref[0] + acc_ref[1] + g_ref[...]) * dinv[:, None] + b_ref[...]
    row = m * _BN + jax.lax.broadcasted_iota(jnp.int32, (_BN, 1), 0)
    oc_ref[...] = jnp.where(row < _N, oc, 0.0)


def _oc_tc(acc, g, degp, b2):
    return pl.pallas_call(
        _oc_body,
        grid=(_NP // _BN,),
        in_specs=[
            pl.BlockSpec((_NC, _BN, _D), lambda m: (0, m, 0)),
            pl.BlockSpec((_BN, _D), lambda m: (m, 0)),
            pl.BlockSpec((_NC, _BN), lambda m: (0, m)),
            pl.BlockSpec((1, _D), lambda m: (0, 0)),
        ],
        out_specs=pl.BlockSpec((_BN, _D), lambda m: (m, 0)),
        out_shape=jax.ShapeDtypeStruct((_NP, _D), jnp.float32),
    )(acc, g, degp, b2)


def _mm_body(origin_ref, oc_ref, out_ref, acc_ref):
    k = pl.program_id(1)
    a = origin_ref[...]
    limit = _N - k * _BK
    col = jax.lax.broadcasted_iota(jnp.int32, a.shape, 1)
    a = jnp.where(col < limit, a, 0.0)
    b = oc_ref[pl.ds(k * _BK, _BK), :]
    part = jnp.dot(a, b, preferred_element_type=jnp.float32)

    @pl.when(k == 0)
    def _():
        acc_ref[...] = jnp.zeros_like(acc_ref)

    acc_ref[...] += part

    @pl.when(k == _NKB - 1)
    def _():
        out_ref[...] = jnp.maximum(acc_ref[...], 0.0)


def _readout_matmul(origin, oc_pad):
    return pl.pallas_call(
        _mm_body,
        grid=(_N // _BM, _NKB),
        in_specs=[
            pl.BlockSpec((_BM, _BK), lambda m, k: (m, k)),
            pl.BlockSpec((_NP, _D), lambda m, k: (0, 0)),
        ],
        out_specs=pl.BlockSpec((_BM, _D), lambda m, k: (m, 0)),
        out_shape=jax.ShapeDtypeStruct((_N, _D), jnp.float32),
        scratch_shapes=[pltpu.VMEM((_BM, _D), jnp.float32)],
        compiler_params=pltpu.CompilerParams(
            dimension_semantics=("parallel", "arbitrary"),
        ),
    )(origin, oc_pad)


def kernel(shuf, origin, i, sparse, edge_index, W, b):
    edge_flat = jnp.ravel(edge_index)
    degp = _deg_kernel(edge_flat)
    g = _g_tc(shuf, W, degp)
    acc = _msg_kernel(g, edge_flat)
    oc_pad = _oc_tc(acc, g, degp, b.reshape(1, _D))
    out = _readout_matmul(origin, oc_pad)
    return out[None]
